# Initial kernel scaffold; baseline (speedup 1.0000x reference)
import jax
import jax.numpy as jnp
from jax.experimental import pallas as pl


def kernel(input_ids, word_embeddings, position_embeddings):
    raise NotImplementedError("write your pallas kernel here")


# speedup vs baseline: 2.0466x; 2.0466x over previous
import functools

import jax
import jax.numpy as jnp
from jax import lax
from jax.experimental import pallas as pl
from jax.experimental.pallas import tpu as pltpu
from jax.experimental.pallas import tpu_sc as plsc

_NC = 2
_NS = 16
_LANES = 16
_CHUNK = 128


def kernel(input_ids, word_embeddings, position_embeddings):
    B, L = input_ids.shape
    V, H = word_embeddings.shape
    N = B * L
    NW = _NC * _NS
    n_chunks = N // _CHUNK
    chunks_per_w = n_chunks // NW
    assert n_chunks * _CHUNK == N and chunks_per_w * NW == n_chunks

    ids2d = input_ids.reshape(n_chunks, _CHUNK)
    pos = position_embeddings[:L]

    mesh = plsc.VectorSubcoreMesh(core_axis_name="c", subcore_axis_name="s")

    @functools.partial(
        pl.kernel,
        out_type=jax.ShapeDtypeStruct((N, H), jnp.float32),
        mesh=mesh,
        scratch_types=[
            pltpu.VMEM((chunks_per_w, _CHUNK), jnp.int32),
            pltpu.VMEM((L, H), jnp.float32),
            pltpu.VMEM((_CHUNK, H), jnp.float32),
            pltpu.SemaphoreType.DMA,
        ],
        compiler_params=pltpu.CompilerParams(use_tc_tiling_on_sc=False),
    )
    def emb_kernel(ids_hbm, tab_hbm, pos_hbm, out_hbm, idx_v, pos_v, buf_v, sem):
        wid = lax.axis_index("s") * _NC + lax.axis_index("c")
        c0 = wid * chunks_per_w
        pltpu.sync_copy(ids_hbm.at[pl.ds(c0, chunks_per_w)], idx_v)
        pltpu.sync_copy(pos_hbm, pos_v)

        def chunk_body(c, carry):
            pltpu.async_copy(tab_hbm.at[idx_v.at[c]], buf_v, sem).wait()
            base = (c0 + c) * _CHUNK
            l0 = lax.rem(base, L)

            def row_body(j, carry2):
                l = lax.rem(l0 + j, L)
                for q in range(H // _LANES):
                    plsc.addupdate(
                        buf_v.at[j, pl.ds(q * _LANES, _LANES)],
                        pos_v[l, pl.ds(q * _LANES, _LANES)],
                    )
                return carry2

            lax.fori_loop(0, _CHUNK, row_body, 0, unroll=4)
            pltpu.sync_copy(buf_v, out_hbm.at[pl.ds(base, _CHUNK)])
            return carry

        lax.fori_loop(0, chunks_per_w, chunk_body, 0)

    out = emb_kernel(ids2d, word_embeddings, pos)
    return out.reshape(B, L, H)


# speedup vs baseline: 2.4358x; 1.1902x over previous
import functools

import jax
import jax.numpy as jnp
from jax import lax
from jax.experimental import pallas as pl
from jax.experimental.pallas import tpu as pltpu
from jax.experimental.pallas import tpu_sc as plsc

_NC = 2
_NS = 16
_LANES = 16
_CHUNK = 128
_NBUF = 8


def kernel(input_ids, word_embeddings, position_embeddings):
    B, L = input_ids.shape
    V, H = word_embeddings.shape
    N = B * L
    NW = _NC * _NS
    n_chunks = N // _CHUNK
    chunks_per_w = n_chunks // NW
    assert n_chunks * _CHUNK == N and chunks_per_w * NW == n_chunks
    assert chunks_per_w % _NBUF == 0
    rounds = chunks_per_w // _NBUF
    lead = _NBUF - 2

    ids2d = input_ids.reshape(n_chunks, _CHUNK)
    pos = position_embeddings[:L]

    mesh = plsc.VectorSubcoreMesh(core_axis_name="c", subcore_axis_name="s")

    @functools.partial(
        pl.kernel,
        out_type=jax.ShapeDtypeStruct((N, H), jnp.float32),
        mesh=mesh,
        scratch_types=[
            pltpu.VMEM((chunks_per_w, _CHUNK), jnp.int32),
            pltpu.VMEM((L, H), jnp.float32),
            [pltpu.VMEM((_CHUNK, H), jnp.float32) for _ in range(_NBUF)],
            [pltpu.SemaphoreType.DMA for _ in range(_NBUF)],
            [pltpu.SemaphoreType.DMA for _ in range(_NBUF)],
        ],
        compiler_params=pltpu.CompilerParams(use_tc_tiling_on_sc=False),
    )
    def emb_kernel(ids_hbm, tab_hbm, pos_hbm, out_hbm, idx_v, pos_v, bufs,
                   gsems, osems):
        wid = lax.axis_index("s") * _NC + lax.axis_index("c")
        c0 = wid * chunks_per_w
        pltpu.sync_copy(ids_hbm.at[pl.ds(c0, chunks_per_w)], idx_v)
        pltpu.sync_copy(pos_hbm, pos_v)

        def start_gather(c, slot):
            pltpu.async_copy(tab_hbm.at[idx_v.at[c]], bufs[slot], gsems[slot])

        def wait_gather(slot):
            pltpu.make_async_copy(out_hbm.at[pl.ds(0, _CHUNK)], bufs[slot],
                                  gsems[slot]).wait()

        def start_out(c, slot):
            base = (c0 + c) * _CHUNK
            pltpu.async_copy(bufs[slot], out_hbm.at[pl.ds(base, _CHUNK)],
                             osems[slot])

        def wait_out(slot):
            pltpu.make_async_copy(bufs[slot], out_hbm.at[pl.ds(0, _CHUNK)],
                                  osems[slot]).wait()

        for b in range(lead):
            start_gather(b, b)

        def compute(c, slot):
            l0 = lax.rem(c * _CHUNK, L)

            def row_body(j, carry):
                l = lax.rem(l0 + j, L)
                for q in range(H // _LANES):
                    plsc.addupdate(
                        bufs[slot].at[j, pl.ds(q * _LANES, _LANES)],
                        pos_v[l, pl.ds(q * _LANES, _LANES)],
                    )
                return carry

            lax.fori_loop(0, _CHUNK, row_body, 0, unroll=4)

        @pl.loop(0, rounds)
        def _round(r):
            for b in range(_NBUF):
                c = r * _NBUF + b
                slot_ahead = (b + lead) % _NBUF
                wait_gather(b)
                compute(c, b)
                start_out(c, b)

                @pl.when(c + lead < chunks_per_w)
                def _():
                    @pl.when(c >= _NBUF - lead)
                    def _():
                        wait_out(slot_ahead)
                    start_gather(c + lead, slot_ahead)

        for b in range(_NBUF):
            wait_out(b)

    out = emb_kernel(ids2d, word_embeddings, pos)
    return out.reshape(B, L, H)
